# baseline (speedup 1.0000x reference)
import functools

import jax
import jax.numpy as jnp
from jax import lax
from jax.experimental import pallas as pl
from jax.experimental.pallas import tpu as pltpu
from jax.experimental.pallas import tpu_sc as plsc

KC = 8192
DC = 64
BETA_C = 0.25

TN = 512
TK = 2048
NB = 16384 // TN
KB = KC // TK


RC = 32
BIGI = 2 ** 30


def _vq_argmin_body(flat2_ref, embt_ref, fnorm_ref, enorm_ref,
                    idx_ref, bval_ref, buf_a, buf_b, bval_s, bidx_s):
    fb2 = flat2_ref[...]
    bufs = [buf_a, buf_b]

    for kb in range(KB):
        buf = bufs[kb % 2]
        buf[...] = lax.dot_general(
            fb2, embt_ref[:, kb * TK:(kb + 1) * TK],
            (((1,), (0,)), ((), ())), preferred_element_type=jnp.float32)
        en_tile = enorm_ref[:, kb * TK:(kb + 1) * TK]
        base = kb * TK

        def row_body(r, _, buf=buf, en_tile=en_tile, base=base, kb=kb):
            r0 = r * RC
            mmc = buf[pl.ds(r0, RC), :]
            fnc = fnorm_ref[pl.ds(r0, RC), :]

            runval = (fnc + en_tile[:, 0:128]) - mmc[:, 0:128]
            runidx = jnp.full((RC, 128), base, jnp.int32)
            for c in range(1, TK // 128):
                s = (fnc + en_tile[:, c * 128:(c + 1) * 128]) \
                    - mmc[:, c * 128:(c + 1) * 128]
                m = s < runval
                runval = jnp.where(m, s, runval)
                runidx = jnp.where(m, jnp.int32(base + c * 128), runidx)

            gmin = jnp.min(runval, axis=1, keepdims=True)
            lane = lax.broadcasted_iota(jnp.int32, (RC, 128), 1)
            cand = jnp.where(runval == gmin, runidx + lane, jnp.int32(BIGI))
            lidx = jnp.min(cand, axis=1, keepdims=True)

            if kb == 0:
                bval_s[pl.ds(r0, RC), :] = gmin
                bidx_s[pl.ds(r0, RC), :] = lidx
            else:
                bv = bval_s[pl.ds(r0, RC), :]
                bt = gmin < bv
                bidx_s[pl.ds(r0, RC), :] = jnp.where(
                    bt, lidx, bidx_s[pl.ds(r0, RC), :])
                bval_s[pl.ds(r0, RC), :] = jnp.where(bt, gmin, bv)
            return 0

        lax.fori_loop(0, TN // RC, row_body, 0)

    idx_ref[0, :, :] = bidx_s[...]
    bval_ref[0, :, :] = bval_s[...]


def _vq_argmin(flat2, embt, fnorm, enorm):
    return pl.pallas_call(
        _vq_argmin_body,
        grid=(NB,),
        in_specs=[
            pl.BlockSpec((TN, DC), lambda i: (i, 0)),
            pl.BlockSpec((DC, KC), lambda i: (0, 0)),
            pl.BlockSpec((TN, 1), lambda i: (i, 0)),
            pl.BlockSpec((1, KC), lambda i: (0, 0)),
        ],
        out_specs=[
            pl.BlockSpec((1, TN, 1), lambda i: (i, 0, 0)),
            pl.BlockSpec((1, TN, 1), lambda i: (i, 0, 0)),
        ],
        out_shape=[
            jax.ShapeDtypeStruct((NB, TN, 1), jnp.int32),
            jax.ShapeDtypeStruct((NB, TN, 1), jnp.float32),
        ],
        scratch_shapes=[
            pltpu.VMEM((TN, TK), jnp.float32),
            pltpu.VMEM((TN, TK), jnp.float32),
            pltpu.VMEM((TN, 1), jnp.float32),
            pltpu.VMEM((TN, 1), jnp.int32),
        ],
        compiler_params=pltpu.CompilerParams(
            dimension_semantics=("arbitrary",)),
    )(flat2, embt, fnorm, enorm)


_SC_GATHER_CACHE = []


def _build_sc_gather():
    info = plsc.get_sparse_core_info()
    nc = info.num_cores
    nw = nc * info.num_subcores
    bpw = 16384 // nw

    @functools.partial(
        pl.kernel,
        mesh=plsc.VectorSubcoreMesh(core_axis_name="c", subcore_axis_name="s"),
        out_type=jax.ShapeDtypeStruct((16384, DC), jnp.float32),
        scratch_types=[
            pltpu.VMEM((bpw,), jnp.int32),
            pltpu.VMEM((bpw, DC), jnp.float32),
            pltpu.SemaphoreType.DMA,
        ],
        compiler_params=pltpu.CompilerParams(use_tc_tiling_on_sc=False),
    )
    def gather_rows(table_hbm, idx_hbm, out_hbm, idx_v, rows_v, sem):
        wid = lax.axis_index("s") * nc + lax.axis_index("c")
        base = wid * bpw
        pltpu.sync_copy(idx_hbm.at[pl.ds(base, bpw)], idx_v)
        pltpu.async_copy(table_hbm.at[idx_v], rows_v, sem).wait()
        pltpu.sync_copy(rows_v, out_hbm.at[pl.ds(base, bpw)])

    return gather_rows


def _sc_gather_rows(table, indices):
    if not _SC_GATHER_CACHE:
        _SC_GATHER_CACHE.append(_build_sc_gather())
    return _SC_GATHER_CACHE[0](table, indices)


def kernel(z_e, embedding):
    B, Dc, H, W = z_e.shape
    N = B * H * W
    flat = jnp.transpose(z_e, (0, 2, 3, 1)).reshape(N, Dc)
    fnorm = jnp.sum(flat ** 2, axis=1, keepdims=True)
    enorm = jnp.sum(embedding ** 2, axis=1).reshape(1, KC)
    embt = embedding.T

    idx3, bv3 = _vq_argmin(flat + flat, embt, fnorm, enorm)
    indices = idx3.reshape(N)

    z_q_flat = _sc_gather_rows(embedding, indices)
    z_q = jnp.transpose(z_q_flat.reshape(B, H, W, Dc), (0, 3, 1, 2))

    commitment_loss = BETA_C * (jnp.sum(bv3) / (N * Dc))
    return (z_q, indices.reshape(B, H, W), commitment_loss)


# speedup vs baseline: 4.1018x; 4.1018x over previous
import functools

import jax
import jax.numpy as jnp
from jax import lax
from jax.experimental import pallas as pl
from jax.experimental.pallas import tpu as pltpu
from jax.experimental.pallas import tpu_sc as plsc

KC = 8192
DC = 64
BETA_C = 0.25

TN = 512
TK = 2048
NB = 16384 // TN
KB = KC // TK


RC = 128
BIGI = 2 ** 30


def _vq_argmin_body(flat2_ref, embt_ref, fnorm_ref, enorm_ref,
                    idx_ref, bval_ref, buf_a, buf_b, bval_s, bidx_s):
    fb2 = flat2_ref[...]
    bufs = [buf_a, buf_b]

    for kb in range(KB):
        buf = bufs[kb % 2]
        buf[...] = lax.dot_general(
            fb2, embt_ref[:, kb * TK:(kb + 1) * TK],
            (((1,), (0,)), ((), ())), preferred_element_type=jnp.float32)
        en_tile = enorm_ref[:, kb * TK:(kb + 1) * TK]
        base = kb * TK

        for r in range(TN // RC):
            r0 = r * RC
            mmc = buf[r0:r0 + RC, :]
            fnc = fnorm_ref[r0:r0 + RC, :]

            runval = (fnc + en_tile[:, 0:128]) - mmc[:, 0:128]
            runidx = jnp.full((RC, 128), base, jnp.int32)
            for c in range(1, TK // 128):
                s = (fnc + en_tile[:, c * 128:(c + 1) * 128]) \
                    - mmc[:, c * 128:(c + 1) * 128]
                m = s < runval
                runval = jnp.where(m, s, runval)
                runidx = jnp.where(m, jnp.int32(base + c * 128), runidx)

            gmin = jnp.min(runval, axis=1, keepdims=True)
            lane = lax.broadcasted_iota(jnp.int32, (RC, 128), 1)
            cand = jnp.where(runval == gmin, runidx + lane, jnp.int32(BIGI))
            lidx = jnp.min(cand, axis=1, keepdims=True)

            if kb == 0:
                bval_s[r0:r0 + RC, :] = gmin
                bidx_s[r0:r0 + RC, :] = lidx
            else:
                bv = bval_s[r0:r0 + RC, :]
                bt = gmin < bv
                bidx_s[r0:r0 + RC, :] = jnp.where(
                    bt, lidx, bidx_s[r0:r0 + RC, :])
                bval_s[r0:r0 + RC, :] = jnp.where(bt, gmin, bv)

    idx_ref[0, :, :] = bidx_s[...]
    bval_ref[0, :, :] = bval_s[...]


def _vq_argmin(flat2, embt, fnorm, enorm):
    return pl.pallas_call(
        _vq_argmin_body,
        grid=(NB,),
        in_specs=[
            pl.BlockSpec((TN, DC), lambda i: (i, 0)),
            pl.BlockSpec((DC, KC), lambda i: (0, 0)),
            pl.BlockSpec((TN, 1), lambda i: (i, 0)),
            pl.BlockSpec((1, KC), lambda i: (0, 0)),
        ],
        out_specs=[
            pl.BlockSpec((1, TN, 1), lambda i: (i, 0, 0)),
            pl.BlockSpec((1, TN, 1), lambda i: (i, 0, 0)),
        ],
        out_shape=[
            jax.ShapeDtypeStruct((NB, TN, 1), jnp.int32),
            jax.ShapeDtypeStruct((NB, TN, 1), jnp.float32),
        ],
        scratch_shapes=[
            pltpu.VMEM((TN, TK), jnp.float32),
            pltpu.VMEM((TN, TK), jnp.float32),
            pltpu.VMEM((TN, 1), jnp.float32),
            pltpu.VMEM((TN, 1), jnp.int32),
        ],
        compiler_params=pltpu.CompilerParams(
            dimension_semantics=("arbitrary",)),
    )(flat2, embt, fnorm, enorm)


_SC_GATHER_CACHE = []


def _build_sc_gather():
    info = plsc.get_sparse_core_info()
    nc = info.num_cores
    nw = nc * info.num_subcores
    bpw = 16384 // nw

    @functools.partial(
        pl.kernel,
        mesh=plsc.VectorSubcoreMesh(core_axis_name="c", subcore_axis_name="s"),
        out_type=jax.ShapeDtypeStruct((16384, DC), jnp.float32),
        scratch_types=[
            pltpu.VMEM((bpw,), jnp.int32),
            pltpu.VMEM((bpw, DC), jnp.float32),
            pltpu.SemaphoreType.DMA,
        ],
        compiler_params=pltpu.CompilerParams(use_tc_tiling_on_sc=False),
    )
    def gather_rows(table_hbm, idx_hbm, out_hbm, idx_v, rows_v, sem):
        wid = lax.axis_index("s") * nc + lax.axis_index("c")
        base = wid * bpw
        pltpu.sync_copy(idx_hbm.at[pl.ds(base, bpw)], idx_v)
        pltpu.async_copy(table_hbm.at[idx_v], rows_v, sem).wait()
        pltpu.sync_copy(rows_v, out_hbm.at[pl.ds(base, bpw)])

    return gather_rows


def _sc_gather_rows(table, indices):
    if not _SC_GATHER_CACHE:
        _SC_GATHER_CACHE.append(_build_sc_gather())
    return _SC_GATHER_CACHE[0](table, indices)


def kernel(z_e, embedding):
    B, Dc, H, W = z_e.shape
    N = B * H * W
    flat = jnp.transpose(z_e, (0, 2, 3, 1)).reshape(N, Dc)
    fnorm = jnp.sum(flat ** 2, axis=1, keepdims=True)
    enorm = jnp.sum(embedding ** 2, axis=1).reshape(1, KC)
    embt = embedding.T

    idx3, bv3 = _vq_argmin(flat + flat, embt, fnorm, enorm)
    indices = idx3.reshape(N)

    z_q_flat = _sc_gather_rows(embedding, indices)
    z_q = jnp.transpose(z_q_flat.reshape(B, H, W, Dc), (0, 3, 1, 2))

    commitment_loss = BETA_C * (jnp.sum(bv3) / (N * Dc))
    return (z_q, indices.reshape(B, H, W), commitment_loss)


# speedup vs baseline: 4.2582x; 1.0381x over previous
import functools

import jax
import jax.numpy as jnp
from jax import lax
from jax.experimental import pallas as pl
from jax.experimental.pallas import tpu as pltpu
from jax.experimental.pallas import tpu_sc as plsc

KC = 8192
DC = 64
BETA_C = 0.25

TN = 1024
TK = 2048
NB = 16384 // TN
KB = KC // TK


RC = 128
BIGI = 2 ** 30


def _vq_argmin_body(flat2_ref, embt_ref, fnorm_ref, enorm_ref,
                    idx_ref, bval_ref, buf_a, buf_b, bval_s, bidx_s):
    fb2 = flat2_ref[...]
    bufs = [buf_a, buf_b]

    for kb in range(KB):
        buf = bufs[kb % 2]
        buf[...] = lax.dot_general(
            fb2, embt_ref[:, kb * TK:(kb + 1) * TK],
            (((1,), (0,)), ((), ())), preferred_element_type=jnp.float32)
        en_tile = enorm_ref[:, kb * TK:(kb + 1) * TK]
        base = kb * TK

        for r in range(TN // RC):
            r0 = r * RC
            mmc = buf[r0:r0 + RC, :]
            fnc = fnorm_ref[r0:r0 + RC, :]

            runval = (fnc + en_tile[:, 0:128]) - mmc[:, 0:128]
            runidx = jnp.full((RC, 128), base, jnp.int32)
            for c in range(1, TK // 128):
                s = (fnc + en_tile[:, c * 128:(c + 1) * 128]) \
                    - mmc[:, c * 128:(c + 1) * 128]
                m = s < runval
                runval = jnp.where(m, s, runval)
                runidx = jnp.where(m, jnp.int32(base + c * 128), runidx)

            gmin = jnp.min(runval, axis=1, keepdims=True)
            lane = lax.broadcasted_iota(jnp.int32, (RC, 128), 1)
            cand = jnp.where(runval == gmin, runidx + lane, jnp.int32(BIGI))
            lidx = jnp.min(cand, axis=1, keepdims=True)

            if kb == 0:
                bval_s[r0:r0 + RC, :] = gmin
                bidx_s[r0:r0 + RC, :] = lidx
            else:
                bv = bval_s[r0:r0 + RC, :]
                bt = gmin < bv
                bidx_s[r0:r0 + RC, :] = jnp.where(
                    bt, lidx, bidx_s[r0:r0 + RC, :])
                bval_s[r0:r0 + RC, :] = jnp.where(bt, gmin, bv)

    idx_ref[0, :, :] = bidx_s[...]
    bval_ref[0, :, :] = bval_s[...]


def _vq_argmin(flat2, embt, fnorm, enorm):
    return pl.pallas_call(
        _vq_argmin_body,
        grid=(NB,),
        in_specs=[
            pl.BlockSpec((TN, DC), lambda i: (i, 0)),
            pl.BlockSpec((DC, KC), lambda i: (0, 0)),
            pl.BlockSpec((TN, 1), lambda i: (i, 0)),
            pl.BlockSpec((1, KC), lambda i: (0, 0)),
        ],
        out_specs=[
            pl.BlockSpec((1, TN, 1), lambda i: (i, 0, 0)),
            pl.BlockSpec((1, TN, 1), lambda i: (i, 0, 0)),
        ],
        out_shape=[
            jax.ShapeDtypeStruct((NB, TN, 1), jnp.int32),
            jax.ShapeDtypeStruct((NB, TN, 1), jnp.float32),
        ],
        scratch_shapes=[
            pltpu.VMEM((TN, TK), jnp.float32),
            pltpu.VMEM((TN, TK), jnp.float32),
            pltpu.VMEM((TN, 1), jnp.float32),
            pltpu.VMEM((TN, 1), jnp.int32),
        ],
        compiler_params=pltpu.CompilerParams(
            dimension_semantics=("arbitrary",)),
    )(flat2, embt, fnorm, enorm)


_SC_GATHER_CACHE = []


def _build_sc_gather():
    info = plsc.get_sparse_core_info()
    nc = info.num_cores
    nw = nc * info.num_subcores
    bpw = 16384 // nw

    @functools.partial(
        pl.kernel,
        mesh=plsc.VectorSubcoreMesh(core_axis_name="c", subcore_axis_name="s"),
        out_type=jax.ShapeDtypeStruct((16384, DC), jnp.float32),
        scratch_types=[
            pltpu.VMEM((bpw,), jnp.int32),
            pltpu.VMEM((bpw, DC), jnp.float32),
            pltpu.SemaphoreType.DMA,
        ],
        compiler_params=pltpu.CompilerParams(use_tc_tiling_on_sc=False),
    )
    def gather_rows(table_hbm, idx_hbm, out_hbm, idx_v, rows_v, sem):
        wid = lax.axis_index("s") * nc + lax.axis_index("c")
        base = wid * bpw
        pltpu.sync_copy(idx_hbm.at[pl.ds(base, bpw)], idx_v)
        pltpu.async_copy(table_hbm.at[idx_v], rows_v, sem).wait()
        pltpu.sync_copy(rows_v, out_hbm.at[pl.ds(base, bpw)])

    return gather_rows


def _sc_gather_rows(table, indices):
    if not _SC_GATHER_CACHE:
        _SC_GATHER_CACHE.append(_build_sc_gather())
    return _SC_GATHER_CACHE[0](table, indices)


def kernel(z_e, embedding):
    B, Dc, H, W = z_e.shape
    N = B * H * W
    flat = jnp.transpose(z_e, (0, 2, 3, 1)).reshape(N, Dc)
    fnorm = jnp.sum(flat ** 2, axis=1, keepdims=True)
    enorm = jnp.sum(embedding ** 2, axis=1).reshape(1, KC)
    embt = embedding.T

    idx3, bv3 = _vq_argmin(flat + flat, embt, fnorm, enorm)
    indices = idx3.reshape(N)

    z_q_flat = _sc_gather_rows(embedding, indices)
    z_q = jnp.transpose(z_q_flat.reshape(B, H, W, Dc), (0, 3, 1, 2))

    commitment_loss = BETA_C * (jnp.sum(bv3) / (N * Dc))
    return (z_q, indices.reshape(B, H, W), commitment_loss)


# speedup vs baseline: 4.3446x; 1.0203x over previous
import functools

import jax
import jax.numpy as jnp
from jax import lax
from jax.experimental import pallas as pl
from jax.experimental.pallas import tpu as pltpu
from jax.experimental.pallas import tpu_sc as plsc

KC = 8192
DC = 64
BETA_C = 0.25

TN = 1024
TK = 2048
NB = 16384 // TN
KB = KC // TK


RC = 128
BIGI = 2 ** 30


def _vq_argmin_body(flat2_ref, embt_ref, fnorm_ref, enorm_ref,
                    idx_ref, lsum_ref, buf_a, buf_b, bval_s, bidx_s):
    fb2 = flat2_ref[...]
    bufs = [buf_a, buf_b]

    def issue_dot(kb):
        bufs[kb % 2][...] = lax.dot_general(
            fb2, embt_ref[:, kb * TK:(kb + 1) * TK],
            (((1,), (0,)), ((), ())), preferred_element_type=jnp.float32)

    def reduce_tile(kb):
        buf = bufs[kb % 2]
        en_tile = enorm_ref[:, kb * TK:(kb + 1) * TK]
        base = kb * TK

        for r in range(TN // RC):
            r0 = r * RC
            mmc = buf[r0:r0 + RC, :]
            fnc = fnorm_ref[r0:r0 + RC, :]

            runval = (fnc + en_tile[:, 0:128]) - mmc[:, 0:128]
            runidx = jnp.full((RC, 128), base, jnp.int32)
            for c in range(1, TK // 128):
                s = (fnc + en_tile[:, c * 128:(c + 1) * 128]) \
                    - mmc[:, c * 128:(c + 1) * 128]
                m = s < runval
                runval = jnp.where(m, s, runval)
                runidx = jnp.where(m, jnp.int32(base + c * 128), runidx)

            gmin = jnp.min(runval, axis=1, keepdims=True)
            lane = lax.broadcasted_iota(jnp.int32, (RC, 128), 1)
            cand = jnp.where(runval == gmin, runidx + lane, jnp.int32(BIGI))
            lidx = jnp.min(cand, axis=1, keepdims=True)

            if kb == 0:
                bval_s[r0:r0 + RC, :] = gmin
                bidx_s[r0:r0 + RC, :] = lidx
            else:
                bv = bval_s[r0:r0 + RC, :]
                bt = gmin < bv
                bidx_s[r0:r0 + RC, :] = jnp.where(
                    bt, lidx, bidx_s[r0:r0 + RC, :])
                bval_s[r0:r0 + RC, :] = jnp.where(bt, gmin, bv)

    issue_dot(0)
    issue_dot(1)
    reduce_tile(0)
    issue_dot(2)
    reduce_tile(1)
    issue_dot(3)
    reduce_tile(2)
    reduce_tile(3)

    idx_ref[0, :, :] = bidx_s[...]

    part = jnp.sum(bval_s[...])
    i = pl.program_id(0)

    @pl.when(i == 0)
    def _():
        lsum_ref[0, 0] = 0.0

    lsum_ref[0, 0] += part


def _vq_argmin(flat2, embt, fnorm, enorm):
    return pl.pallas_call(
        _vq_argmin_body,
        grid=(NB,),
        in_specs=[
            pl.BlockSpec((TN, DC), lambda i: (i, 0)),
            pl.BlockSpec((DC, KC), lambda i: (0, 0)),
            pl.BlockSpec((TN, 1), lambda i: (i, 0)),
            pl.BlockSpec((1, KC), lambda i: (0, 0)),
        ],
        out_specs=[
            pl.BlockSpec((1, TN, 1), lambda i: (i, 0, 0)),
            pl.BlockSpec(memory_space=pltpu.SMEM),
        ],
        out_shape=[
            jax.ShapeDtypeStruct((NB, TN, 1), jnp.int32),
            jax.ShapeDtypeStruct((1, 1), jnp.float32),
        ],
        scratch_shapes=[
            pltpu.VMEM((TN, TK), jnp.float32),
            pltpu.VMEM((TN, TK), jnp.float32),
            pltpu.VMEM((TN, 1), jnp.float32),
            pltpu.VMEM((TN, 1), jnp.int32),
        ],
        compiler_params=pltpu.CompilerParams(
            dimension_semantics=("arbitrary",)),
    )(flat2, embt, fnorm, enorm)


_SC_GATHER_CACHE = []


def _build_sc_gather():
    info = plsc.get_sparse_core_info()
    nc = info.num_cores
    nw = nc * info.num_subcores
    bpw = 16384 // nw

    half = bpw // 2

    @functools.partial(
        pl.kernel,
        mesh=plsc.VectorSubcoreMesh(core_axis_name="c", subcore_axis_name="s"),
        out_type=jax.ShapeDtypeStruct((16384, DC), jnp.float32),
        scratch_types=[
            pltpu.VMEM((half,), jnp.int32),
            pltpu.VMEM((half,), jnp.int32),
            pltpu.VMEM((half, DC), jnp.float32),
            pltpu.VMEM((half, DC), jnp.float32),
            pltpu.SemaphoreType.DMA,
            pltpu.SemaphoreType.DMA,
        ],
        compiler_params=pltpu.CompilerParams(use_tc_tiling_on_sc=False),
    )
    def gather_rows(table_hbm, idx_hbm, out_hbm,
                    idx_a, idx_b, rows_a, rows_b, gsem, wsem):
        wid = lax.axis_index("s") * nc + lax.axis_index("c")
        base = wid * bpw
        pltpu.sync_copy(idx_hbm.at[pl.ds(base, half)], idx_a)
        pltpu.sync_copy(idx_hbm.at[pl.ds(base + half, half)], idx_b)
        pltpu.async_copy(table_hbm.at[idx_a], rows_a, gsem).wait()
        wb_a = pltpu.async_copy(rows_a, out_hbm.at[pl.ds(base, half)], wsem)
        pltpu.async_copy(table_hbm.at[idx_b], rows_b, gsem).wait()
        pltpu.sync_copy(rows_b, out_hbm.at[pl.ds(base + half, half)])
        wb_a.wait()

    return gather_rows


def _sc_gather_rows(table, indices):
    if not _SC_GATHER_CACHE:
        _SC_GATHER_CACHE.append(_build_sc_gather())
    return _SC_GATHER_CACHE[0](table, indices)


def kernel(z_e, embedding):
    B, Dc, H, W = z_e.shape
    N = B * H * W
    flat = jnp.transpose(z_e, (0, 2, 3, 1)).reshape(N, Dc)
    fnorm = jnp.sum(flat ** 2, axis=1, keepdims=True)
    enorm = jnp.sum(embedding ** 2, axis=1).reshape(1, KC)
    embt = embedding.T

    idx3, lsum = _vq_argmin(flat + flat, embt, fnorm, enorm)
    indices = idx3.reshape(N)

    z_q_flat = _sc_gather_rows(embedding, indices)
    z_q = jnp.transpose(z_q_flat.reshape(B, H, W, Dc), (0, 3, 1, 2))

    commitment_loss = BETA_C * (lsum[0, 0] / (N * Dc))
    return (z_q, indices.reshape(B, H, W), commitment_loss)


# speedup vs baseline: 4.4382x; 1.0215x over previous
import functools

import jax
import jax.numpy as jnp
from jax import lax
from jax.experimental import pallas as pl
from jax.experimental.pallas import tpu as pltpu
from jax.experimental.pallas import tpu_sc as plsc

KC = 8192
DC = 64
BETA_C = 0.25

TN = 2048
TK = 2048
NB = 16384 // TN
KB = KC // TK


RC = 128
BIGI = 2 ** 30


def _vq_argmin_body(flat2_ref, embt_ref, fnorm_ref, enorm_ref,
                    idx_ref, lsum_ref, buf_a, buf_b, bval_s, bidx_s):
    fb = flat2_ref[...]
    fb2 = fb + fb
    bufs = [buf_a, buf_b]

    def issue_dot(kb):
        bufs[kb % 2][...] = lax.dot_general(
            fb2, embt_ref[:, kb * TK:(kb + 1) * TK],
            (((1,), (0,)), ((), ())), preferred_element_type=jnp.float32)

    def reduce_tile(kb):
        buf = bufs[kb % 2]
        en_tile = enorm_ref[:, kb * TK:(kb + 1) * TK]
        base = kb * TK

        for r in range(TN // RC):
            r0 = r * RC
            mmc = buf[r0:r0 + RC, :]
            fnc = fnorm_ref[r0:r0 + RC, :]

            runval = (fnc + en_tile[:, 0:128]) - mmc[:, 0:128]
            runidx = jnp.full((RC, 128), base, jnp.int32)
            for c in range(1, TK // 128):
                s = (fnc + en_tile[:, c * 128:(c + 1) * 128]) \
                    - mmc[:, c * 128:(c + 1) * 128]
                m = s < runval
                runval = jnp.where(m, s, runval)
                runidx = jnp.where(m, jnp.int32(base + c * 128), runidx)

            gmin = jnp.min(runval, axis=1, keepdims=True)
            lane = lax.broadcasted_iota(jnp.int32, (RC, 128), 1)
            cand = jnp.where(runval == gmin, runidx + lane, jnp.int32(BIGI))
            lidx = jnp.min(cand, axis=1, keepdims=True)

            if kb == 0:
                bval_s[r0:r0 + RC, :] = gmin
                bidx_s[r0:r0 + RC, :] = lidx
            else:
                bv = bval_s[r0:r0 + RC, :]
                bt = gmin < bv
                bidx_s[r0:r0 + RC, :] = jnp.where(
                    bt, lidx, bidx_s[r0:r0 + RC, :])
                bval_s[r0:r0 + RC, :] = jnp.where(bt, gmin, bv)

    issue_dot(0)
    issue_dot(1)
    reduce_tile(0)
    issue_dot(2)
    reduce_tile(1)
    issue_dot(3)
    reduce_tile(2)
    reduce_tile(3)

    idx_ref[0, :, :] = bidx_s[...]

    part = jnp.sum(bval_s[...])
    i = pl.program_id(0)

    @pl.when(i == 0)
    def _():
        lsum_ref[0, 0] = 0.0

    lsum_ref[0, 0] += part


def _vq_argmin(flat2, embt, fnorm, enorm):
    return pl.pallas_call(
        _vq_argmin_body,
        grid=(NB,),
        in_specs=[
            pl.BlockSpec((TN, DC), lambda i: (i, 0)),
            pl.BlockSpec((DC, KC), lambda i: (0, 0)),
            pl.BlockSpec((TN, 1), lambda i: (i, 0)),
            pl.BlockSpec((1, KC), lambda i: (0, 0)),
        ],
        out_specs=[
            pl.BlockSpec((1, TN, 1), lambda i: (i, 0, 0)),
            pl.BlockSpec(memory_space=pltpu.SMEM),
        ],
        out_shape=[
            jax.ShapeDtypeStruct((NB, TN, 1), jnp.int32),
            jax.ShapeDtypeStruct((1, 1), jnp.float32),
        ],
        scratch_shapes=[
            pltpu.VMEM((TN, TK), jnp.float32),
            pltpu.VMEM((TN, TK), jnp.float32),
            pltpu.VMEM((TN, 1), jnp.float32),
            pltpu.VMEM((TN, 1), jnp.int32),
        ],
        compiler_params=pltpu.CompilerParams(
            dimension_semantics=("arbitrary",)),
    )(flat2, embt, fnorm, enorm)


_SC_GATHER_CACHE = []


def _build_sc_gather():
    info = plsc.get_sparse_core_info()
    nc = info.num_cores
    nw = nc * info.num_subcores
    bpw = 16384 // nw

    half = bpw // 2

    @functools.partial(
        pl.kernel,
        mesh=plsc.VectorSubcoreMesh(core_axis_name="c", subcore_axis_name="s"),
        out_type=jax.ShapeDtypeStruct((16384, DC), jnp.float32),
        scratch_types=[
            pltpu.VMEM((half,), jnp.int32),
            pltpu.VMEM((half,), jnp.int32),
            pltpu.VMEM((half, DC), jnp.float32),
            pltpu.VMEM((half, DC), jnp.float32),
            pltpu.SemaphoreType.DMA,
            pltpu.SemaphoreType.DMA,
        ],
        compiler_params=pltpu.CompilerParams(use_tc_tiling_on_sc=False),
    )
    def gather_rows(table_hbm, idx_hbm, out_hbm,
                    idx_a, idx_b, rows_a, rows_b, gsem, wsem):
        wid = lax.axis_index("s") * nc + lax.axis_index("c")
        base = wid * bpw
        pltpu.sync_copy(idx_hbm.at[pl.ds(base, half)], idx_a)
        pltpu.sync_copy(idx_hbm.at[pl.ds(base + half, half)], idx_b)
        pltpu.async_copy(table_hbm.at[idx_a], rows_a, gsem).wait()
        wb_a = pltpu.async_copy(rows_a, out_hbm.at[pl.ds(base, half)], wsem)
        pltpu.async_copy(table_hbm.at[idx_b], rows_b, gsem).wait()
        pltpu.sync_copy(rows_b, out_hbm.at[pl.ds(base + half, half)])
        wb_a.wait()

    return gather_rows


def _sc_gather_rows(table, indices):
    if not _SC_GATHER_CACHE:
        _SC_GATHER_CACHE.append(_build_sc_gather())
    return _SC_GATHER_CACHE[0](table, indices)


def kernel(z_e, embedding):
    B, Dc, H, W = z_e.shape
    N = B * H * W
    flat = jnp.transpose(z_e, (0, 2, 3, 1)).reshape(N, Dc)
    fnorm = jnp.sum(flat ** 2, axis=1, keepdims=True)
    enorm = jnp.sum(embedding ** 2, axis=1).reshape(1, KC)
    embt = embedding.T

    idx3, lsum = _vq_argmin(flat, embt, fnorm, enorm)
    indices = idx3.reshape(N)

    z_q_flat = _sc_gather_rows(embedding, indices)
    z_q = jnp.transpose(z_q_flat.reshape(B, H, W, Dc), (0, 3, 1, 2))

    commitment_loss = BETA_C * (lsum[0, 0] / (N * Dc))
    return (z_q, indices.reshape(B, H, W), commitment_loss)


# speedup vs baseline: 4.4430x; 1.0011x over previous
import functools

import jax
import jax.numpy as jnp
from jax import lax
from jax.experimental import pallas as pl
from jax.experimental.pallas import tpu as pltpu
from jax.experimental.pallas import tpu_sc as plsc

KC = 8192
DC = 64
BETA_C = 0.25

TN = 2048
TK = 2048
NB = 16384 // TN
KB = KC // TK


RC = 128
BIGI = 2 ** 30


def _vq_argmin_body(flat_ref, embt_ref, fnorm_ref, enorm_ref,
                    idx_ref, lsum_ref, buf_a, buf_b, bval_s, bidx_s):
    fb = flat_ref[...]
    fb2 = fb + fb
    bufs = [buf_a, buf_b]

    def issue_dot(kb):
        bufs[kb % 2][...] = lax.dot_general(
            fb2, embt_ref[:, kb * TK:(kb + 1) * TK],
            (((1,), (0,)), ((), ())), preferred_element_type=jnp.float32)

    def reduce_tile(kb):
        buf = bufs[kb % 2]
        en_tile = enorm_ref[:, kb * TK:(kb + 1) * TK]
        base = kb * TK

        for r in range(TN // RC):
            r0 = r * RC
            mmc = buf[r0:r0 + RC, :]
            fnc = fnorm_ref[r0:r0 + RC, :]

            runval = (fnc + en_tile[:, 0:128]) - mmc[:, 0:128]
            runidx = jnp.full((RC, 128), base, jnp.int32)
            for c in range(1, TK // 128):
                s = (fnc + en_tile[:, c * 128:(c + 1) * 128]) \
                    - mmc[:, c * 128:(c + 1) * 128]
                m = s < runval
                runval = jnp.where(m, s, runval)
                runidx = jnp.where(m, jnp.int32(base + c * 128), runidx)

            gmin = jnp.min(runval, axis=1, keepdims=True)
            lane = lax.broadcasted_iota(jnp.int32, (RC, 128), 1)
            cand = jnp.where(runval == gmin, runidx + lane, jnp.int32(BIGI))
            lidx = jnp.min(cand, axis=1, keepdims=True)

            if kb == 0:
                bval_s[r0:r0 + RC, :] = gmin
                bidx_s[r0:r0 + RC, :] = lidx
            else:
                bv = bval_s[r0:r0 + RC, :]
                bt = gmin < bv
                bidx_s[r0:r0 + RC, :] = jnp.where(
                    bt, lidx, bidx_s[r0:r0 + RC, :])
                bval_s[r0:r0 + RC, :] = jnp.where(bt, gmin, bv)

    issue_dot(0)
    issue_dot(1)
    for kb in range(KB - 2):
        reduce_tile(kb)
        issue_dot(kb + 2)
    reduce_tile(KB - 2)
    reduce_tile(KB - 1)

    idx_ref[0, :, :] = bidx_s[...]

    part = jnp.sum(bval_s[...])
    i = pl.program_id(0)

    @pl.when(i == 0)
    def _():
        lsum_ref[0, 0] = 0.0

    lsum_ref[0, 0] += part


def _vq_argmin(ze3, emb, fnorm, enorm):
    return pl.pallas_call(
        _vq_argmin_body,
        grid=(NB,),
        in_specs=[
            pl.BlockSpec((TN, DC), lambda i: (i, 0)),
            pl.BlockSpec((DC, KC), lambda i: (0, 0)),
            pl.BlockSpec((TN, 1), lambda i: (i, 0)),
            pl.BlockSpec((1, KC), lambda i: (0, 0)),
        ],
        out_specs=[
            pl.BlockSpec((1, TN, 1), lambda i: (i, 0, 0)),
            pl.BlockSpec(memory_space=pltpu.SMEM),
        ],
        out_shape=[
            jax.ShapeDtypeStruct((NB, TN, 1), jnp.int32),
            jax.ShapeDtypeStruct((1, 1), jnp.float32),
        ],
        scratch_shapes=[
            pltpu.VMEM((TN, TK), jnp.float32),
            pltpu.VMEM((TN, TK), jnp.float32),
            pltpu.VMEM((TN, 1), jnp.float32),
            pltpu.VMEM((TN, 1), jnp.int32),
        ],
        compiler_params=pltpu.CompilerParams(
            dimension_semantics=("arbitrary",)),
    )(ze3, emb, fnorm, enorm)


_SC_GATHER_CACHE = []


def _build_sc_gather():
    info = plsc.get_sparse_core_info()
    nc = info.num_cores
    nw = nc * info.num_subcores
    bpw = 16384 // nw

    half = bpw // 2

    @functools.partial(
        pl.kernel,
        mesh=plsc.VectorSubcoreMesh(core_axis_name="c", subcore_axis_name="s"),
        out_type=jax.ShapeDtypeStruct((16384, DC), jnp.float32),
        scratch_types=[
            pltpu.VMEM((half,), jnp.int32),
            pltpu.VMEM((half,), jnp.int32),
            pltpu.VMEM((half, DC), jnp.float32),
            pltpu.VMEM((half, DC), jnp.float32),
            pltpu.SemaphoreType.DMA,
            pltpu.SemaphoreType.DMA,
        ],
        compiler_params=pltpu.CompilerParams(use_tc_tiling_on_sc=False),
    )
    def gather_rows(table_hbm, idx_hbm, out_hbm,
                    idx_a, idx_b, rows_a, rows_b, gsem, wsem):
        wid = lax.axis_index("s") * nc + lax.axis_index("c")
        base = wid * bpw
        pltpu.sync_copy(idx_hbm.at[pl.ds(base, half)], idx_a)
        pltpu.sync_copy(idx_hbm.at[pl.ds(base + half, half)], idx_b)
        pltpu.async_copy(table_hbm.at[idx_a], rows_a, gsem).wait()
        wb_a = pltpu.async_copy(rows_a, out_hbm.at[pl.ds(base, half)], wsem)
        pltpu.async_copy(table_hbm.at[idx_b], rows_b, gsem).wait()
        pltpu.sync_copy(rows_b, out_hbm.at[pl.ds(base + half, half)])
        wb_a.wait()

    return gather_rows


def _sc_gather_rows(table, indices):
    if not _SC_GATHER_CACHE:
        _SC_GATHER_CACHE.append(_build_sc_gather())
    return _SC_GATHER_CACHE[0](table, indices)


def kernel(z_e, embedding):
    B, Dc, H, W = z_e.shape
    N = B * H * W
    flat = jnp.transpose(z_e, (0, 2, 3, 1)).reshape(N, Dc)
    fnorm = jnp.sum(flat ** 2, axis=1, keepdims=True)
    enorm = jnp.sum(embedding ** 2, axis=1).reshape(1, KC)
    embt = embedding.T

    idx3, lsum = _vq_argmin(flat, embt, fnorm, enorm)
    indices = idx3.reshape(N)

    z_q_flat = _sc_gather_rows(embedding, indices)
    z_q = jnp.transpose(z_q_flat.reshape(B, H, W, Dc), (0, 3, 1, 2))

    commitment_loss = BETA_C * (lsum[0, 0] / (N * Dc))
    return (z_q, indices.reshape(B, H, W), commitment_loss)
